# baseline (speedup 1.0000x reference)
import functools

import jax
import jax.numpy as jnp
from jax import lax
from jax.experimental import pallas as pl
from jax.experimental.pallas import tpu as pltpu
from jax.experimental.pallas import tpu_sc as plsc

DELTA = 2.5
NC, NS, NW = 2, 16, 32
N = 10000
NP = 10240
DUMP = NP - 1
E = 160000
EP = 163840


def _mm_body(x_ref, w_ref, b_ref, o_ref):
    o_ref[...] = (
        jnp.dot(x_ref[...], w_ref[...], preferred_element_type=jnp.float32)
        + b_ref[...]
    )


def _mm(x, w, b=None, mt=512):
    m, k = x.shape
    nc = w.shape[1]
    if b is None:
        b = jnp.zeros((nc,), jnp.float32)
    b2 = b.reshape(1, nc)
    grid = pl.cdiv(m, mt)
    return pl.pallas_call(
        _mm_body,
        grid=(grid,),
        in_specs=[
            pl.BlockSpec((mt, k), lambda i: (i, 0)),
            pl.BlockSpec((k, nc), lambda i: (0, 0)),
            pl.BlockSpec((1, nc), lambda i: (0, 0)),
        ],
        out_specs=pl.BlockSpec((mt, nc), lambda i: (i, 0)),
        out_shape=jax.ShapeDtypeStruct((m, nc), jnp.float32),
    )(x, w, b2)


def _pad_k(x, w, mult=8):
    k = x.shape[1]
    kp = (-k) % mult
    if kp:
        x = jnp.pad(x, ((0, 0), (0, kp)))
        w = jnp.pad(w, ((0, kp), (0, 0)))
    return x, w


@functools.partial(jax.jit, static_argnums=(2,))
def _sc_gather(table, idx2d, ep):
    t, d = table.shape
    ki = 2 if d == 128 else 1
    per_w = ep // NW
    rps = ki * 128
    nsuper = per_w // rps
    assert per_w % rps == 0 and nsuper % 2 == 0 and d in (128, 256)

    @functools.partial(
        pl.kernel,
        out_type=jax.ShapeDtypeStruct((ep, d), jnp.float32),
        mesh=plsc.VectorSubcoreMesh(core_axis_name="c", subcore_axis_name="s"),
        scratch_types=[
            pltpu.VMEM((ki, 128), jnp.int32),
            pltpu.VMEM((ki, 128), jnp.int32),
            pltpu.VMEM((rps, d), jnp.float32),
            pltpu.VMEM((rps, d), jnp.float32),
            pltpu.SemaphoreType.DMA,
            pltpu.SemaphoreType.DMA,
            pltpu.SemaphoreType.DMA,
        ],
    )
    def k(table_hbm, idx_hbm, out_hbm, idx0, idx1, rows0, rows1,
          sem_g, sem_o0, sem_o1):
        wid = lax.axis_index("s") * NC + lax.axis_index("c")
        base_i = wid * (per_w // 128)
        base_e = wid * per_w
        bufs = ((idx0, rows0, sem_o0), (idx1, rows1, sem_o1))

        def sbody(i2, carry):
            for b in range(2):
                idx_v, rows_v, sem_o = bufs[b]
                i = i2 * 2 + b
                pltpu.sync_copy(idx_hbm.at[pl.ds(base_i + i * ki, ki)], idx_v)

                @pl.when(i2 > 0)
                def _():
                    pltpu.make_async_copy(
                        rows_v, out_hbm.at[pl.ds(base_e, rps)], sem_o).wait()

                cps = [
                    pltpu.async_copy(table_hbm.at[idx_v.at[j]],
                                     rows_v.at[pl.ds(j * 128, 128)], sem_g)
                    for j in range(ki)
                ]
                for cp in cps:
                    cp.wait()
                pltpu.async_copy(
                    rows_v, out_hbm.at[pl.ds(base_e + i * rps, rps)], sem_o)
            return carry

        lax.fori_loop(0, nsuper // 2, sbody, 0)
        for b in range(2):
            idx_v, rows_v, sem_o = bufs[b]
            pltpu.make_async_copy(
                rows_v, out_hbm.at[pl.ds(base_e, rps)], sem_o).wait()

    return k(table, idx2d)


NH = 5120
NACC = 5248


@jax.jit
def _sc_scatter_add_wide(rows, idxl2d, zeros):
    ep, d = rows.shape
    dc = 128
    ki = 2
    nchunk = d // dc
    per_t = ep // NS
    rps = ki * 128
    nsuper = per_t // rps
    rpt = NACC // NS
    assert per_t % rps == 0 and d % dc == 0 and nsuper % 2 == 0

    @functools.partial(
        pl.kernel,
        out_type=jax.ShapeDtypeStruct((NC, NACC, d), jnp.float32),
        mesh=plsc.VectorSubcoreMesh(core_axis_name="c", subcore_axis_name="s"),
        scratch_types=[
            pltpu.VMEM((ki, 128), jnp.int32),
            pltpu.VMEM((ki, 128), jnp.int32),
            pltpu.VMEM((rps, dc), jnp.float32),
            pltpu.VMEM((rps, dc), jnp.float32),
            pltpu.SemaphoreType.DMA,
            pltpu.SemaphoreType.DMA,
            pltpu.VMEM_SHARED((NACC, dc), jnp.float32),
        ],
    )
    def k(rows_hbm, idx_hbm, zeros_hbm, out_hbm, idx0, idx1, rows0, rows1,
          sem0, sem1, acc):
        cid = lax.axis_index("c")
        sid = lax.axis_index("s")
        base_i = sid * (per_t // 128)
        base_e = sid * per_t
        bufs = ((idx0, rows0, sem0), (idx1, rows1, sem1))

        def stage(i, b, c):
            idx_v, rows_v, sem = bufs[b]
            pltpu.async_copy(
                rows_hbm.at[pl.ds(base_e + i * rps, rps),
                            pl.ds(c * dc, dc)],
                rows_v, sem)

        for c in range(nchunk):
            pltpu.sync_copy(zeros_hbm, acc.at[pl.ds(sid * rpt, rpt)])
            plsc.subcore_barrier()
            stage(0, 0, c)
            stage(1, 1, c)

            def sbody(i2, carry):
                for b in range(2):
                    idx_v, rows_v, sem = bufs[b]
                    i = i2 * 2 + b
                    pltpu.sync_copy(
                        idx_hbm.at[cid, pl.ds(base_i + i * ki, ki)], idx_v)
                    pltpu.make_async_copy(
                        rows_hbm.at[pl.ds(base_e, rps), pl.ds(0, dc)],
                        rows_v, sem).wait()
                    for j in range(ki):
                        pltpu.sync_copy(rows_v.at[pl.ds(j * 128, 128)],
                                        acc.at[idx_v.at[j]], add=True)

                    @pl.when(i + 2 < nsuper)
                    def _():
                        stage(i + 2, b, c)
                return carry

            lax.fori_loop(0, nsuper // 2, sbody, 0)
            plsc.subcore_barrier()
            pltpu.sync_copy(
                acc.at[pl.ds(sid * rpt, rpt)],
                out_hbm.at[cid, pl.ds(sid * rpt, rpt), pl.ds(c * dc, dc)])
            if c + 1 < nchunk:
                plsc.subcore_barrier()

    return k(rows, idxl2d, zeros)


NEG = -1.0e30


@jax.jit
def _sc_seg_max4(es4, srcg, dstg):
    per_w = EP // NW
    rps = 512
    nsuper = per_w // rps

    @functools.partial(
        pl.kernel,
        out_type=jax.ShapeDtypeStruct((NW, NP * 4), jnp.float32),
        mesh=plsc.VectorSubcoreMesh(core_axis_name="c", subcore_axis_name="s"),
        scratch_types=[
            pltpu.VMEM((N * 4 + 16,), jnp.float32),
            pltpu.VMEM((rps,), jnp.int32),
            pltpu.VMEM((rps,), jnp.int32),
            pltpu.VMEM((NP * 4 + 16,), jnp.float32),
        ],
    )
    def k(es_hbm, src_hbm, dst_hbm, out_hbm, es_v, src_v, dst_v, acc):
        wid = lax.axis_index("s") * NC + lax.axis_index("c")
        base_e = wid * per_w
        neg = jnp.full((16,), NEG, jnp.float32)

        def init(i, carry):
            acc[pl.ds(i * 16, 16)] = neg
            return carry

        lax.fori_loop(0, (NP * 4 + 16) // 16, init, 0)
        pltpu.sync_copy(es_hbm, es_v)

        lane = lax.iota(jnp.int32, 16)
        msk4 = lane < 4

        def sbody(i, carry):
            pltpu.sync_copy(src_hbm.at[pl.ds(base_e + i * rps, rps)], src_v)
            pltpu.sync_copy(dst_hbm.at[pl.ds(base_e + i * rps, rps)], dst_v)

            def gbody(g, carry2):
                svec = src_v[pl.ds(g * 16, 16)]
                dvec = dst_v[pl.ds(g * 16, 16)]
                for l in range(16):
                    s = svec[l]
                    d = dvec[l]
                    val = es_v[pl.ds(s * 4, 16)]
                    cur = acc[pl.ds(d * 4, 16)]
                    acc[pl.ds(d * 4, 16)] = jnp.where(
                        msk4, jnp.maximum(cur, val), cur)
                return carry2

            lax.fori_loop(0, rps // 16, gbody, 0)
            return carry

        lax.fori_loop(0, nsuper, sbody, 0)
        pltpu.sync_copy(acc.at[pl.ds(0, NP * 4)], out_hbm.at[wid])

    part = k(es4, srcg, dstg)
    return jnp.max(part, axis=0).reshape(NP, 4)[:N]


def _seg_max8(es8, srcg, dstg):
    outs = []
    for p in range(2):
        es4 = jnp.pad(es8[:, 4 * p:4 * p + 4].reshape(-1), (0, 16))
        outs.append(_sc_seg_max4(es4, srcg, dstg))
    return jnp.concatenate(outs, axis=1)


def _seg_sum(rows, idxl2d, zeros):
    part = _sc_scatter_add_wide(rows, idxl2d, zeros)
    return jnp.concatenate([part[0, :NH], part[1, :N - NH]], axis=0)


def _pna(h, srcg2d, dstg2d, dstl2d, dst, zerosw, ef_p, W_pna, b_pna):
    hsrc = _sc_gather(h, srcg2d, EP)
    m = jax.nn.relu(hsrc + ef_p)
    s = _seg_sum(m, dstl2d, zerosw)
    sq = _seg_sum(m * m, dstl2d, zerosw)
    ones = jnp.zeros((EP, 128), jnp.float32).at[:, 0].set(1.0)
    deg = _seg_sum(ones, dstl2d, zerosw)[:, 0]
    degc = jnp.clip(deg, 1.0, None)[:, None]
    mean = s / degc
    mx = jax.ops.segment_max(m[:E], dst, num_segments=N)
    mx = jnp.where(jnp.isfinite(mx), mx, 0.0)
    var = jnp.clip(sq / degc - mean * mean, 0.0, None)
    std = jnp.sqrt(var + 1e-5)
    aggr = jnp.concatenate([mean, mx, s, std], axis=-1)
    slog = jnp.log(deg + 1.0)[:, None]
    amp = slog / DELTA
    att = DELTA / jnp.clip(slog, 1e-5, None)
    scaled = jnp.concatenate([aggr, aggr * amp, aggr * att], axis=-1)
    return _mm(scaled, W_pna, b_pna, mt=256)


def _gat(x, srcg, srcg2d, dstg2d, dstl2d, dsts, idx22d, idx42d, zerosw,
         W, a_s, a_d, Ws, b, heads, fh, concat, activate):
    n = x.shape[0]
    hf = heads * fh
    hw = _mm(x, jnp.concatenate([W, Ws], axis=1), mt=512)
    h = hw[:, :hf]
    hs = hw[:, hf:]
    h3 = h.reshape(n, heads, fh)
    es = jnp.sum(h3 * a_s[None, :, :], axis=-1)
    ed = jnp.sum(h3 * a_d[None, :, :], axis=-1)
    es8 = jnp.zeros((n, 8), jnp.float32).at[:, 0:heads].set(es)
    smx = _seg_max8(es8, srcg, dsts)[:, 0:heads]
    emax = jnp.where(smx < -1e29, 0.0,
                     jax.nn.leaky_relu(smx + ed, 0.2))
    esed = jnp.zeros((n, 128), jnp.float32)
    esed = esed.at[:, 0:heads].set(es).at[:, 8:8 + heads].set(ed)
    esed = esed.at[:, 16:16 + heads].set(emax)
    gs = _sc_gather(esed, srcg2d, EP)
    gd = _sc_gather(esed, dstg2d, EP)
    e = jax.nn.leaky_relu(gs[:, 0:heads] + gd[:, 8:8 + heads], 0.2)
    ee = jnp.exp(e - gd[:, 16:16 + heads])
    ee128 = jnp.zeros((EP, 128), jnp.float32).at[:, 0:heads].set(ee)
    dent = _seg_sum(ee128, dstl2d, zerosw)
    den_g = _sc_gather(dent, dstg2d, EP)[:, 0:heads]
    alpha = ee / (den_g + 1e-16)
    if hf == 1024:
        hsrc = _sc_gather(h.reshape(4 * n, 256), idx42d, 4 * EP).reshape(
            EP, hf)
    elif hf == 512:
        hsrc = _sc_gather(h.reshape(2 * n, 256), idx22d, 2 * EP).reshape(
            EP, hf)
    else:
        hsrc = _sc_gather(h, srcg2d, EP)
    weighted = (alpha[:, :, None] * hsrc.reshape(EP, heads, fh)).reshape(
        EP, hf)
    out = _seg_sum(weighted, dstl2d, zerosw).reshape(n, heads, fh)
    out = out + hs.reshape(n, heads, fh)
    if concat:
        out = out.reshape(n, hf)
    else:
        out = out.mean(axis=1)
    out = out + b
    if activate:
        out = jax.nn.elu(out)
    return out


def kernel(task_fea, mach_fea, edge_index, edge_fea, W_task, b_task, W_mach,
           b_mach, W_epna, W_pna, b_pna, W0, as0, ad0, Ws0, b0, W1, as1, ad1,
           Ws1, b1, W2, as2, ad2, Ws2, b2):
    src = edge_index[0]
    dst = edge_index[1]
    padlen = EP - E
    srcg = jnp.concatenate([src, jnp.zeros((padlen,), jnp.int32)])
    dstg = jnp.concatenate([dst, jnp.zeros((padlen,), jnp.int32)])
    dsts = jnp.concatenate([dst, jnp.full((padlen,), DUMP, jnp.int32)])
    srcg2d = srcg.reshape(EP // 128, 128)
    dstg2d = dstg.reshape(EP // 128, 128)
    idx2 = jnp.stack([2 * src, 2 * src + 1], axis=1).reshape(-1)
    idx2 = jnp.concatenate([idx2, jnp.zeros((2 * padlen,), jnp.int32)])
    idx22d = idx2.reshape(2 * EP // 128, 128)
    idx4 = (4 * src[:, None] + jnp.arange(4, dtype=jnp.int32)[None, :]
            ).reshape(-1)
    idx4 = jnp.concatenate([idx4, jnp.zeros((4 * padlen,), jnp.int32)])
    idx42d = idx4.reshape(4 * EP // 128, 128)
    zerosw = jnp.zeros((NACC // NS, 128), jnp.float32)
    dstl = []
    for c in range(NC):
        lo = c * NH
        inr = (dsts >= lo) & (dsts < lo + NH)
        dstl.append(jnp.where(inr, dsts - lo, NH))
    dstl2d = jnp.stack(dstl).reshape(NC, EP // 128, 128)

    tf = _mm(*_pad_k(task_fea, W_task), b_task, mt=1000)
    mf = _mm(*_pad_k(mach_fea, W_mach), b_mach, mt=1000)
    node_fea = jnp.concatenate([tf, mf], axis=0)

    ef_p = _mm(jnp.pad(edge_fea, ((0, padlen), (0, 0))), W_epna, mt=2048)
    aggr = _pna(node_fea, srcg2d, dstg2d, dstl2d, dst, zerosw, ef_p,
                W_pna, b_pna)
    h0 = _gat(aggr, srcg, srcg2d, dstg2d, dstl2d, dsts, idx22d, idx42d, zerosw,
              W0, as0, ad0, Ws0, b0, 8, 64, True, True)
    h1 = _gat(h0, srcg, srcg2d, dstg2d, dstl2d, dsts, idx22d, idx42d, zerosw,
              W1, as1, ad1, Ws1, b1, 8, 128, True, True)
    h2 = _gat(h1, srcg, srcg2d, dstg2d, dstl2d, dsts, idx22d, idx42d, zerosw,
              W2, as2, ad2, Ws2, b2, 1, 256, False, False)
    return h2


# speedup vs baseline: 1.0153x; 1.0153x over previous
import functools

import jax
import jax.numpy as jnp
from jax import lax
from jax.experimental import pallas as pl
from jax.experimental.pallas import tpu as pltpu
from jax.experimental.pallas import tpu_sc as plsc

DELTA = 2.5
NC, NS, NW = 2, 16, 32
N = 10000
NP = 10240
DUMP = NP - 1
E = 160000
EP = 163840


def _mm_body(x_ref, w_ref, b_ref, o_ref):
    o_ref[...] = (
        jnp.dot(x_ref[...], w_ref[...], preferred_element_type=jnp.float32)
        + b_ref[...]
    )


def _mm(x, w, b=None, mt=512):
    m, k = x.shape
    nc = w.shape[1]
    if b is None:
        b = jnp.zeros((nc,), jnp.float32)
    b2 = b.reshape(1, nc)
    grid = pl.cdiv(m, mt)
    return pl.pallas_call(
        _mm_body,
        grid=(grid,),
        in_specs=[
            pl.BlockSpec((mt, k), lambda i: (i, 0)),
            pl.BlockSpec((k, nc), lambda i: (0, 0)),
            pl.BlockSpec((1, nc), lambda i: (0, 0)),
        ],
        out_specs=pl.BlockSpec((mt, nc), lambda i: (i, 0)),
        out_shape=jax.ShapeDtypeStruct((m, nc), jnp.float32),
    )(x, w, b2)


def _pad_k(x, w, mult=8):
    k = x.shape[1]
    kp = (-k) % mult
    if kp:
        x = jnp.pad(x, ((0, 0), (0, kp)))
        w = jnp.pad(w, ((0, kp), (0, 0)))
    return x, w


@functools.partial(jax.jit, static_argnums=(2,))
def _sc_gather(table, idx2d, ep):
    t, d = table.shape
    ki = 2 if d == 128 else 1
    per_w = ep // NW
    rps = ki * 128
    nsuper = per_w // rps
    assert per_w % rps == 0 and nsuper % 2 == 0 and d in (128, 256)

    @functools.partial(
        pl.kernel,
        out_type=jax.ShapeDtypeStruct((ep, d), jnp.float32),
        mesh=plsc.VectorSubcoreMesh(core_axis_name="c", subcore_axis_name="s"),
        scratch_types=[
            pltpu.VMEM((ki, 128), jnp.int32),
            pltpu.VMEM((ki, 128), jnp.int32),
            pltpu.VMEM((rps, d), jnp.float32),
            pltpu.VMEM((rps, d), jnp.float32),
            pltpu.SemaphoreType.DMA,
            pltpu.SemaphoreType.DMA,
            pltpu.SemaphoreType.DMA,
            pltpu.SemaphoreType.DMA,
        ],
    )
    def k(table_hbm, idx_hbm, out_hbm, idx0, idx1, rows0, rows1,
          sem_g0, sem_g1, sem_o0, sem_o1):
        wid = lax.axis_index("s") * NC + lax.axis_index("c")
        base_i = wid * (per_w // 128)
        base_e = wid * per_w
        bufs = ((idx0, rows0, sem_g0, sem_o0), (idx1, rows1, sem_g1, sem_o1))

        def sbody(i2, carry):
            cps = [None, None]
            for b in range(2):
                idx_v, rows_v, sem_g, sem_o = bufs[b]
                i = i2 * 2 + b
                pltpu.sync_copy(idx_hbm.at[pl.ds(base_i + i * ki, ki)], idx_v)

                @pl.when(i2 > 0)
                def _():
                    pltpu.make_async_copy(
                        rows_v, out_hbm.at[pl.ds(base_e, rps)], sem_o).wait()

                cps[b] = [
                    pltpu.async_copy(table_hbm.at[idx_v.at[j]],
                                     rows_v.at[pl.ds(j * 128, 128)], sem_g)
                    for j in range(ki)
                ]
            for b in range(2):
                idx_v, rows_v, sem_g, sem_o = bufs[b]
                i = i2 * 2 + b
                for cp in cps[b]:
                    cp.wait()
                pltpu.async_copy(
                    rows_v, out_hbm.at[pl.ds(base_e + i * rps, rps)], sem_o)
            return carry

        lax.fori_loop(0, nsuper // 2, sbody, 0)
        for b in range(2):
            idx_v, rows_v, sem_g, sem_o = bufs[b]
            pltpu.make_async_copy(
                rows_v, out_hbm.at[pl.ds(base_e, rps)], sem_o).wait()

    return k(table, idx2d)


NH = 5120
NACC = 5248


@jax.jit
def _sc_scatter_add_wide(rows, idxl2d, zeros):
    ep, d = rows.shape
    dc = 128
    ki = 2
    nchunk = d // dc
    per_t = ep // NS
    rps = ki * 128
    nsuper = per_t // rps
    rpt = NACC // NS
    assert per_t % rps == 0 and d % dc == 0 and nsuper % 2 == 0

    @functools.partial(
        pl.kernel,
        out_type=jax.ShapeDtypeStruct((NC, NACC, d), jnp.float32),
        mesh=plsc.VectorSubcoreMesh(core_axis_name="c", subcore_axis_name="s"),
        scratch_types=[
            pltpu.VMEM((ki, 128), jnp.int32),
            pltpu.VMEM((ki, 128), jnp.int32),
            pltpu.VMEM((rps, dc), jnp.float32),
            pltpu.VMEM((rps, dc), jnp.float32),
            pltpu.SemaphoreType.DMA,
            pltpu.SemaphoreType.DMA,
            pltpu.VMEM_SHARED((NACC, dc), jnp.float32),
        ],
    )
    def k(rows_hbm, idx_hbm, zeros_hbm, out_hbm, idx0, idx1, rows0, rows1,
          sem0, sem1, acc):
        cid = lax.axis_index("c")
        sid = lax.axis_index("s")
        base_i = sid * (per_t // 128)
        base_e = sid * per_t
        bufs = ((idx0, rows0, sem0), (idx1, rows1, sem1))

        def stage(i, b, c):
            idx_v, rows_v, sem = bufs[b]
            pltpu.async_copy(
                rows_hbm.at[pl.ds(base_e + i * rps, rps),
                            pl.ds(c * dc, dc)],
                rows_v, sem)

        for c in range(nchunk):
            pltpu.sync_copy(zeros_hbm, acc.at[pl.ds(sid * rpt, rpt)])
            plsc.subcore_barrier()
            stage(0, 0, c)
            stage(1, 1, c)

            def sbody(i2, carry):
                for b in range(2):
                    idx_v, rows_v, sem = bufs[b]
                    i = i2 * 2 + b
                    pltpu.sync_copy(
                        idx_hbm.at[cid, pl.ds(base_i + i * ki, ki)], idx_v)
                    pltpu.make_async_copy(
                        rows_hbm.at[pl.ds(base_e, rps), pl.ds(0, dc)],
                        rows_v, sem).wait()
                    for j in range(ki):
                        pltpu.sync_copy(rows_v.at[pl.ds(j * 128, 128)],
                                        acc.at[idx_v.at[j]], add=True)

                    @pl.when(i + 2 < nsuper)
                    def _():
                        stage(i + 2, b, c)
                return carry

            lax.fori_loop(0, nsuper // 2, sbody, 0)
            plsc.subcore_barrier()
            pltpu.sync_copy(
                acc.at[pl.ds(sid * rpt, rpt)],
                out_hbm.at[cid, pl.ds(sid * rpt, rpt), pl.ds(c * dc, dc)])
            if c + 1 < nchunk:
                plsc.subcore_barrier()

    return k(rows, idxl2d, zeros)


NEG = -1.0e30


@jax.jit
def _sc_seg_max4(es4, srcg, dstg):
    per_w = EP // NW
    rps = 512
    nsuper = per_w // rps

    @functools.partial(
        pl.kernel,
        out_type=jax.ShapeDtypeStruct((NW, NP * 4), jnp.float32),
        mesh=plsc.VectorSubcoreMesh(core_axis_name="c", subcore_axis_name="s"),
        scratch_types=[
            pltpu.VMEM((N * 4 + 16,), jnp.float32),
            pltpu.VMEM((rps,), jnp.int32),
            pltpu.VMEM((rps,), jnp.int32),
            pltpu.VMEM((NP * 4 + 16,), jnp.float32),
        ],
    )
    def k(es_hbm, src_hbm, dst_hbm, out_hbm, es_v, src_v, dst_v, acc):
        wid = lax.axis_index("s") * NC + lax.axis_index("c")
        base_e = wid * per_w
        neg = jnp.full((16,), NEG, jnp.float32)

        def init(i, carry):
            acc[pl.ds(i * 16, 16)] = neg
            return carry

        lax.fori_loop(0, (NP * 4 + 16) // 16, init, 0)
        pltpu.sync_copy(es_hbm, es_v)

        lane = lax.iota(jnp.int32, 16)
        msk4 = lane < 4

        def sbody(i, carry):
            pltpu.sync_copy(src_hbm.at[pl.ds(base_e + i * rps, rps)], src_v)
            pltpu.sync_copy(dst_hbm.at[pl.ds(base_e + i * rps, rps)], dst_v)

            def gbody(g, carry2):
                svec = src_v[pl.ds(g * 16, 16)]
                dvec = dst_v[pl.ds(g * 16, 16)]
                for l in range(16):
                    s = svec[l]
                    d = dvec[l]
                    val = es_v[pl.ds(s * 4, 16)]
                    cur = acc[pl.ds(d * 4, 16)]
                    acc[pl.ds(d * 4, 16)] = jnp.where(
                        msk4, jnp.maximum(cur, val), cur)
                return carry2

            lax.fori_loop(0, rps // 16, gbody, 0)
            return carry

        lax.fori_loop(0, nsuper, sbody, 0)
        pltpu.sync_copy(acc.at[pl.ds(0, NP * 4)], out_hbm.at[wid])

    part = k(es4, srcg, dstg)
    return jnp.max(part, axis=0).reshape(NP, 4)[:N]


def _seg_max8(es8, srcg, dstg):
    outs = []
    for p in range(2):
        es4 = jnp.pad(es8[:, 4 * p:4 * p + 4].reshape(-1), (0, 16))
        outs.append(_sc_seg_max4(es4, srcg, dstg))
    return jnp.concatenate(outs, axis=1)


def _seg_sum(rows, idxl2d, zeros):
    part = _sc_scatter_add_wide(rows, idxl2d, zeros)
    return jnp.concatenate([part[0, :NH], part[1, :N - NH]], axis=0)


def _pna(h, srcg2d, dstg2d, dstl2d, dst, zerosw, ef_p, W_pna, b_pna):
    hsrc = _sc_gather(h, srcg2d, EP)
    m = jax.nn.relu(hsrc + ef_p)
    s = _seg_sum(m, dstl2d, zerosw)
    sq = _seg_sum(m * m, dstl2d, zerosw)
    ones = jnp.zeros((EP, 128), jnp.float32).at[:, 0].set(1.0)
    deg = _seg_sum(ones, dstl2d, zerosw)[:, 0]
    degc = jnp.clip(deg, 1.0, None)[:, None]
    mean = s / degc
    mx = jax.ops.segment_max(m[:E], dst, num_segments=N)
    mx = jnp.where(jnp.isfinite(mx), mx, 0.0)
    var = jnp.clip(sq / degc - mean * mean, 0.0, None)
    std = jnp.sqrt(var + 1e-5)
    aggr = jnp.concatenate([mean, mx, s, std], axis=-1)
    slog = jnp.log(deg + 1.0)[:, None]
    amp = slog / DELTA
    att = DELTA / jnp.clip(slog, 1e-5, None)
    scaled = jnp.concatenate([aggr, aggr * amp, aggr * att], axis=-1)
    return _mm(scaled, W_pna, b_pna, mt=256)


def _gat(x, srcg, srcg2d, dstg2d, dstl2d, dsts, idx22d, idx42d, zerosw,
         W, a_s, a_d, Ws, b, heads, fh, concat, activate):
    n = x.shape[0]
    hf = heads * fh
    hw = _mm(x, jnp.concatenate([W, Ws], axis=1), mt=512)
    h = hw[:, :hf]
    hs = hw[:, hf:]
    h3 = h.reshape(n, heads, fh)
    es = jnp.sum(h3 * a_s[None, :, :], axis=-1)
    ed = jnp.sum(h3 * a_d[None, :, :], axis=-1)
    es8 = jnp.zeros((n, 8), jnp.float32).at[:, 0:heads].set(es)
    smx = _seg_max8(es8, srcg, dsts)[:, 0:heads]
    emax = jnp.where(smx < -1e29, 0.0,
                     jax.nn.leaky_relu(smx + ed, 0.2))
    esed = jnp.zeros((n, 128), jnp.float32)
    esed = esed.at[:, 0:heads].set(es).at[:, 8:8 + heads].set(ed)
    esed = esed.at[:, 16:16 + heads].set(emax)
    gs = _sc_gather(esed, srcg2d, EP)
    gd = _sc_gather(esed, dstg2d, EP)
    e = jax.nn.leaky_relu(gs[:, 0:heads] + gd[:, 8:8 + heads], 0.2)
    ee = jnp.exp(e - gd[:, 16:16 + heads])
    ee128 = jnp.zeros((EP, 128), jnp.float32).at[:, 0:heads].set(ee)
    dent = _seg_sum(ee128, dstl2d, zerosw)
    den_g = _sc_gather(dent, dstg2d, EP)[:, 0:heads]
    alpha = ee / (den_g + 1e-16)
    if hf == 1024:
        hsrc = _sc_gather(h.reshape(4 * n, 256), idx42d, 4 * EP).reshape(
            EP, hf)
    elif hf == 512:
        hsrc = _sc_gather(h.reshape(2 * n, 256), idx22d, 2 * EP).reshape(
            EP, hf)
    else:
        hsrc = _sc_gather(h, srcg2d, EP)
    weighted = (alpha[:, :, None] * hsrc.reshape(EP, heads, fh)).reshape(
        EP, hf)
    out = _seg_sum(weighted, dstl2d, zerosw).reshape(n, heads, fh)
    out = out + hs.reshape(n, heads, fh)
    if concat:
        out = out.reshape(n, hf)
    else:
        out = out.mean(axis=1)
    out = out + b
    if activate:
        out = jax.nn.elu(out)
    return out


def kernel(task_fea, mach_fea, edge_index, edge_fea, W_task, b_task, W_mach,
           b_mach, W_epna, W_pna, b_pna, W0, as0, ad0, Ws0, b0, W1, as1, ad1,
           Ws1, b1, W2, as2, ad2, Ws2, b2):
    src = edge_index[0]
    dst = edge_index[1]
    padlen = EP - E
    srcg = jnp.concatenate([src, jnp.zeros((padlen,), jnp.int32)])
    dstg = jnp.concatenate([dst, jnp.zeros((padlen,), jnp.int32)])
    dsts = jnp.concatenate([dst, jnp.full((padlen,), DUMP, jnp.int32)])
    srcg2d = srcg.reshape(EP // 128, 128)
    dstg2d = dstg.reshape(EP // 128, 128)
    idx2 = jnp.stack([2 * src, 2 * src + 1], axis=1).reshape(-1)
    idx2 = jnp.concatenate([idx2, jnp.zeros((2 * padlen,), jnp.int32)])
    idx22d = idx2.reshape(2 * EP // 128, 128)
    idx4 = (4 * src[:, None] + jnp.arange(4, dtype=jnp.int32)[None, :]
            ).reshape(-1)
    idx4 = jnp.concatenate([idx4, jnp.zeros((4 * padlen,), jnp.int32)])
    idx42d = idx4.reshape(4 * EP // 128, 128)
    zerosw = jnp.zeros((NACC // NS, 128), jnp.float32)
    dstl = []
    for c in range(NC):
        lo = c * NH
        inr = (dsts >= lo) & (dsts < lo + NH)
        dstl.append(jnp.where(inr, dsts - lo, NH))
    dstl2d = jnp.stack(dstl).reshape(NC, EP // 128, 128)

    tf = _mm(*_pad_k(task_fea, W_task), b_task, mt=1000)
    mf = _mm(*_pad_k(mach_fea, W_mach), b_mach, mt=1000)
    node_fea = jnp.concatenate([tf, mf], axis=0)

    ef_p = _mm(jnp.pad(edge_fea, ((0, padlen), (0, 0))), W_epna, mt=2048)
    aggr = _pna(node_fea, srcg2d, dstg2d, dstl2d, dst, zerosw, ef_p,
                W_pna, b_pna)
    h0 = _gat(aggr, srcg, srcg2d, dstg2d, dstl2d, dsts, idx22d, idx42d, zerosw,
              W0, as0, ad0, Ws0, b0, 8, 64, True, True)
    h1 = _gat(h0, srcg, srcg2d, dstg2d, dstl2d, dsts, idx22d, idx42d, zerosw,
              W1, as1, ad1, Ws1, b1, 8, 128, True, True)
    h2 = _gat(h1, srcg, srcg2d, dstg2d, dstl2d, dsts, idx22d, idx42d, zerosw,
              W2, as2, ad2, Ws2, b2, 1, 256, False, False)
    return h2


# speedup vs baseline: 1.0221x; 1.0067x over previous
import functools

import jax
import jax.numpy as jnp
from jax import lax
from jax.experimental import pallas as pl
from jax.experimental.pallas import tpu as pltpu
from jax.experimental.pallas import tpu_sc as plsc

DELTA = 2.5
NC, NS, NW = 2, 16, 32
N = 10000
NP = 10240
DUMP = NP - 1
E = 160000
EP = 163840


def _mm_body(x_ref, w_ref, b_ref, o_ref):
    o_ref[...] = (
        jnp.dot(x_ref[...], w_ref[...], preferred_element_type=jnp.float32)
        + b_ref[...]
    )


def _mm(x, w, b=None, mt=512):
    m, k = x.shape
    nc = w.shape[1]
    if b is None:
        b = jnp.zeros((nc,), jnp.float32)
    b2 = b.reshape(1, nc)
    grid = pl.cdiv(m, mt)
    return pl.pallas_call(
        _mm_body,
        grid=(grid,),
        in_specs=[
            pl.BlockSpec((mt, k), lambda i: (i, 0)),
            pl.BlockSpec((k, nc), lambda i: (0, 0)),
            pl.BlockSpec((1, nc), lambda i: (0, 0)),
        ],
        out_specs=pl.BlockSpec((mt, nc), lambda i: (i, 0)),
        out_shape=jax.ShapeDtypeStruct((m, nc), jnp.float32),
    )(x, w, b2)


def _pad_k(x, w, mult=8):
    k = x.shape[1]
    kp = (-k) % mult
    if kp:
        x = jnp.pad(x, ((0, 0), (0, kp)))
        w = jnp.pad(w, ((0, kp), (0, 0)))
    return x, w


@functools.partial(jax.jit, static_argnums=(2,))
def _sc_gather(table, idx2d, ep):
    t, d = table.shape
    ki = 2 if d == 128 else 1
    per_w = ep // NW
    rps = ki * 128
    nsuper = per_w // rps
    assert per_w % rps == 0 and nsuper % 2 == 0 and d in (128, 256)

    @functools.partial(
        pl.kernel,
        out_type=jax.ShapeDtypeStruct((ep, d), jnp.float32),
        mesh=plsc.VectorSubcoreMesh(core_axis_name="c", subcore_axis_name="s"),
        scratch_types=[
            pltpu.VMEM((ki, 128), jnp.int32),
            pltpu.VMEM((ki, 128), jnp.int32),
            pltpu.VMEM((rps, d), jnp.float32),
            pltpu.VMEM((rps, d), jnp.float32),
            pltpu.SemaphoreType.DMA,
            pltpu.SemaphoreType.DMA,
            pltpu.SemaphoreType.DMA,
            pltpu.SemaphoreType.DMA,
        ],
    )
    def k(table_hbm, idx_hbm, out_hbm, idx0, idx1, rows0, rows1,
          sem_g0, sem_g1, sem_o0, sem_o1):
        wid = lax.axis_index("s") * NC + lax.axis_index("c")
        base_i = wid * (per_w // 128)
        base_e = wid * per_w
        bufs = ((idx0, rows0, sem_g0, sem_o0), (idx1, rows1, sem_g1, sem_o1))

        def sbody(i2, carry):
            cps = [None, None]
            for b in range(2):
                idx_v, rows_v, sem_g, sem_o = bufs[b]
                i = i2 * 2 + b
                pltpu.sync_copy(idx_hbm.at[pl.ds(base_i + i * ki, ki)], idx_v)

                @pl.when(i2 > 0)
                def _():
                    pltpu.make_async_copy(
                        rows_v, out_hbm.at[pl.ds(base_e, rps)], sem_o).wait()

                cps[b] = [
                    pltpu.async_copy(table_hbm.at[idx_v.at[j]],
                                     rows_v.at[pl.ds(j * 128, 128)], sem_g)
                    for j in range(ki)
                ]
            for b in range(2):
                idx_v, rows_v, sem_g, sem_o = bufs[b]
                i = i2 * 2 + b
                for cp in cps[b]:
                    cp.wait()
                pltpu.async_copy(
                    rows_v, out_hbm.at[pl.ds(base_e + i * rps, rps)], sem_o)
            return carry

        lax.fori_loop(0, nsuper // 2, sbody, 0)
        for b in range(2):
            idx_v, rows_v, sem_g, sem_o = bufs[b]
            pltpu.make_async_copy(
                rows_v, out_hbm.at[pl.ds(base_e, rps)], sem_o).wait()

    return k(table, idx2d)


NH = 5120
NACC = 5248


@jax.jit
def _sc_scatter_add_wide(rows, idxl2d, zeros):
    ep, d = rows.shape
    dc = 128
    ki = 2
    nchunk = d // dc
    per_t = ep // NS
    rps = ki * 128
    nsuper = per_t // rps
    rpt = NACC // NS
    assert per_t % rps == 0 and d % dc == 0 and nsuper % 2 == 0

    @functools.partial(
        pl.kernel,
        out_type=jax.ShapeDtypeStruct((NC, NACC, d), jnp.float32),
        mesh=plsc.VectorSubcoreMesh(core_axis_name="c", subcore_axis_name="s"),
        scratch_types=[
            pltpu.VMEM((ki, 128), jnp.int32),
            pltpu.VMEM((ki, 128), jnp.int32),
            pltpu.VMEM((rps, dc), jnp.float32),
            pltpu.VMEM((rps, dc), jnp.float32),
            pltpu.SemaphoreType.DMA,
            pltpu.SemaphoreType.DMA,
            pltpu.VMEM_SHARED((NACC, dc), jnp.float32),
        ],
    )
    def k(rows_hbm, idx_hbm, zeros_hbm, out_hbm, idx0, idx1, rows0, rows1,
          sem0, sem1, acc):
        cid = lax.axis_index("c")
        sid = lax.axis_index("s")
        base_i = sid * (per_t // 128)
        base_e = sid * per_t
        bufs = ((idx0, rows0, sem0), (idx1, rows1, sem1))

        def stage(i, b, c):
            idx_v, rows_v, sem = bufs[b]
            pltpu.async_copy(
                rows_hbm.at[pl.ds(base_e + i * rps, rps),
                            pl.ds(c * dc, dc)],
                rows_v, sem)

        for c in range(nchunk):
            pltpu.sync_copy(zeros_hbm, acc.at[pl.ds(sid * rpt, rpt)])
            plsc.subcore_barrier()
            stage(0, 0, c)
            stage(1, 1, c)

            def sbody(i2, carry):
                for b in range(2):
                    idx_v, rows_v, sem = bufs[b]
                    i = i2 * 2 + b
                    pltpu.sync_copy(
                        idx_hbm.at[cid, pl.ds(base_i + i * ki, ki)], idx_v)
                    pltpu.make_async_copy(
                        rows_hbm.at[pl.ds(base_e, rps), pl.ds(0, dc)],
                        rows_v, sem).wait()
                    for j in range(ki):
                        pltpu.sync_copy(rows_v.at[pl.ds(j * 128, 128)],
                                        acc.at[idx_v.at[j]], add=True)

                    @pl.when(i + 2 < nsuper)
                    def _():
                        stage(i + 2, b, c)
                return carry

            lax.fori_loop(0, nsuper // 2, sbody, 0)
            plsc.subcore_barrier()
            pltpu.sync_copy(
                acc.at[pl.ds(sid * rpt, rpt)],
                out_hbm.at[cid, pl.ds(sid * rpt, rpt), pl.ds(c * dc, dc)])
            if c + 1 < nchunk:
                plsc.subcore_barrier()

    return k(rows, idxl2d, zeros)


NEG = -1.0e30


@jax.jit
def _sc_seg_max4(es4, srcg, dstg):
    per_w = EP // NW
    rps = 512
    nsuper = per_w // rps

    @functools.partial(
        pl.kernel,
        out_type=jax.ShapeDtypeStruct((NW, NP * 4), jnp.float32),
        mesh=plsc.VectorSubcoreMesh(core_axis_name="c", subcore_axis_name="s"),
        scratch_types=[
            pltpu.VMEM((N * 4 + 16,), jnp.float32),
            pltpu.VMEM((rps,), jnp.int32),
            pltpu.VMEM((rps,), jnp.int32),
            pltpu.VMEM((NP * 4 + 16,), jnp.float32),
        ],
    )
    def k(es_hbm, src_hbm, dst_hbm, out_hbm, es_v, src_v, dst_v, acc):
        wid = lax.axis_index("s") * NC + lax.axis_index("c")
        base_e = wid * per_w
        neg = jnp.full((16,), NEG, jnp.float32)

        def init(i, carry):
            acc[pl.ds(i * 16, 16)] = neg
            return carry

        lax.fori_loop(0, (NP * 4 + 16) // 16, init, 0)
        pltpu.sync_copy(es_hbm, es_v)

        lane = lax.iota(jnp.int32, 16)
        msk4 = lane < 4

        def sbody(i, carry):
            pltpu.sync_copy(src_hbm.at[pl.ds(base_e + i * rps, rps)], src_v)
            pltpu.sync_copy(dst_hbm.at[pl.ds(base_e + i * rps, rps)], dst_v)

            def gbody(g, carry2):
                svec = src_v[pl.ds(g * 16, 16)]
                dvec = dst_v[pl.ds(g * 16, 16)]
                for l in range(16):
                    s = svec[l]
                    d = dvec[l]
                    val = es_v[pl.ds(s * 4, 16)]
                    cur = acc[pl.ds(d * 4, 16)]
                    acc[pl.ds(d * 4, 16)] = jnp.where(
                        msk4, jnp.maximum(cur, val), cur)
                return carry2

            lax.fori_loop(0, rps // 16, gbody, 0)
            return carry

        lax.fori_loop(0, nsuper, sbody, 0)
        pltpu.sync_copy(acc.at[pl.ds(0, NP * 4)], out_hbm.at[wid])

    part = k(es4, srcg, dstg)
    return jnp.max(part, axis=0).reshape(NP, 4)[:N]


def _seg_max8(es8, srcg, dstg):
    outs = []
    for p in range(2):
        es4 = jnp.pad(es8[:, 4 * p:4 * p + 4].reshape(-1), (0, 16))
        outs.append(_sc_seg_max4(es4, srcg, dstg))
    return jnp.concatenate(outs, axis=1)


def _seg_sum(rows, idxl2d, zeros):
    part = _sc_scatter_add_wide(rows, idxl2d, zeros)
    return jnp.concatenate([part[0, :NH], part[1, :N - NH]], axis=0)


def _pna(h, srcg2d, dstg2d, dstl2d, dsts, zerosw, ef_p, W_pna, b_pna):
    hsrc = _sc_gather(h, srcg2d, EP)
    m = jax.nn.relu(hsrc + ef_p)
    s = _seg_sum(m, dstl2d, zerosw)
    sq = _seg_sum(m * m, dstl2d, zerosw)
    ones = jnp.zeros((EP, 128), jnp.float32).at[:, 0].set(1.0)
    deg = _seg_sum(ones, dstl2d, zerosw)[:, 0]
    degc = jnp.clip(deg, 1.0, None)[:, None]
    mean = s / degc
    mx = jax.ops.segment_max(m, dsts, num_segments=NP)[:N]
    mx = jnp.where(jnp.isfinite(mx), mx, 0.0)
    var = jnp.clip(sq / degc - mean * mean, 0.0, None)
    std = jnp.sqrt(var + 1e-5)
    aggr = jnp.concatenate([mean, mx, s, std], axis=-1)
    slog = jnp.log(deg + 1.0)[:, None]
    amp = slog / DELTA
    att = DELTA / jnp.clip(slog, 1e-5, None)
    scaled = jnp.concatenate([aggr, aggr * amp, aggr * att], axis=-1)
    return _mm(scaled, W_pna, b_pna, mt=256)


def _gat(x, srcg, srcg2d, dstg2d, dstl2d, dsts, idx22d, idx42d, zerosw,
         W, a_s, a_d, Ws, b, heads, fh, concat, activate):
    n = x.shape[0]
    hf = heads * fh
    hw = _mm(x, jnp.concatenate([W, Ws], axis=1), mt=512)
    h = hw[:, :hf]
    hs = hw[:, hf:]
    h3 = h.reshape(n, heads, fh)
    es = jnp.sum(h3 * a_s[None, :, :], axis=-1)
    ed = jnp.sum(h3 * a_d[None, :, :], axis=-1)
    es8 = jnp.zeros((n, 8), jnp.float32).at[:, 0:heads].set(es)
    smx = _seg_max8(es8, srcg, dsts)[:, 0:heads]
    emax = jnp.where(smx < -1e29, 0.0,
                     jax.nn.leaky_relu(smx + ed, 0.2))
    esed = jnp.zeros((n, 128), jnp.float32)
    esed = esed.at[:, 0:heads].set(es).at[:, 8:8 + heads].set(ed)
    esed = esed.at[:, 16:16 + heads].set(emax)
    gs = _sc_gather(esed, srcg2d, EP)
    gd = _sc_gather(esed, dstg2d, EP)
    e = jax.nn.leaky_relu(gs[:, 0:heads] + gd[:, 8:8 + heads], 0.2)
    ee = jnp.exp(e - gd[:, 16:16 + heads])
    ee128 = jnp.zeros((EP, 128), jnp.float32).at[:, 0:heads].set(ee)
    dent = _seg_sum(ee128, dstl2d, zerosw)
    den_g = _sc_gather(dent, dstg2d, EP)[:, 0:heads]
    alpha = ee / (den_g + 1e-16)
    if hf == 1024:
        hsrc = _sc_gather(h.reshape(4 * n, 256), idx42d, 4 * EP).reshape(
            EP, hf)
    elif hf == 512:
        hsrc = _sc_gather(h.reshape(2 * n, 256), idx22d, 2 * EP).reshape(
            EP, hf)
    else:
        hsrc = _sc_gather(h, srcg2d, EP)
    weighted = (alpha[:, :, None] * hsrc.reshape(EP, heads, fh)).reshape(
        EP, hf)
    out = _seg_sum(weighted, dstl2d, zerosw).reshape(n, heads, fh)
    out = out + hs.reshape(n, heads, fh)
    if concat:
        out = out.reshape(n, hf)
    else:
        out = out.mean(axis=1)
    out = out + b
    if activate:
        out = jax.nn.elu(out)
    return out


def kernel(task_fea, mach_fea, edge_index, edge_fea, W_task, b_task, W_mach,
           b_mach, W_epna, W_pna, b_pna, W0, as0, ad0, Ws0, b0, W1, as1, ad1,
           Ws1, b1, W2, as2, ad2, Ws2, b2):
    src = edge_index[0]
    dst = edge_index[1]
    padlen = EP - E
    srcg = jnp.concatenate([src, jnp.zeros((padlen,), jnp.int32)])
    dstg = jnp.concatenate([dst, jnp.zeros((padlen,), jnp.int32)])
    dsts = jnp.concatenate([dst, jnp.full((padlen,), DUMP, jnp.int32)])
    srcg2d = srcg.reshape(EP // 128, 128)
    dstg2d = dstg.reshape(EP // 128, 128)
    idx2 = jnp.stack([2 * src, 2 * src + 1], axis=1).reshape(-1)
    idx2 = jnp.concatenate([idx2, jnp.zeros((2 * padlen,), jnp.int32)])
    idx22d = idx2.reshape(2 * EP // 128, 128)
    idx4 = (4 * src[:, None] + jnp.arange(4, dtype=jnp.int32)[None, :]
            ).reshape(-1)
    idx4 = jnp.concatenate([idx4, jnp.zeros((4 * padlen,), jnp.int32)])
    idx42d = idx4.reshape(4 * EP // 128, 128)
    zerosw = jnp.zeros((NACC // NS, 128), jnp.float32)
    dstl = []
    for c in range(NC):
        lo = c * NH
        inr = (dsts >= lo) & (dsts < lo + NH)
        dstl.append(jnp.where(inr, dsts - lo, NH))
    dstl2d = jnp.stack(dstl).reshape(NC, EP // 128, 128)

    tf = _mm(*_pad_k(task_fea, W_task), b_task, mt=1000)
    mf = _mm(*_pad_k(mach_fea, W_mach), b_mach, mt=1000)
    node_fea = jnp.concatenate([tf, mf], axis=0)

    ef_p = _mm(jnp.pad(edge_fea, ((0, padlen), (0, 0))), W_epna, mt=2048)
    aggr = _pna(node_fea, srcg2d, dstg2d, dstl2d, dsts, zerosw, ef_p,
                W_pna, b_pna)
    h0 = _gat(aggr, srcg, srcg2d, dstg2d, dstl2d, dsts, idx22d, idx42d, zerosw,
              W0, as0, ad0, Ws0, b0, 8, 64, True, True)
    h1 = _gat(h0, srcg, srcg2d, dstg2d, dstl2d, dsts, idx22d, idx42d, zerosw,
              W1, as1, ad1, Ws1, b1, 8, 128, True, True)
    h2 = _gat(h1, srcg, srcg2d, dstg2d, dstl2d, dsts, idx22d, idx42d, zerosw,
              W2, as2, ad2, Ws2, b2, 1, 256, False, False)
    return h2
